# Initial kernel scaffold; baseline (speedup 1.0000x reference)
import jax
import jax.numpy as jnp
from jax.experimental import pallas as pl


def kernel(x, embed_table):
    raise NotImplementedError("write your pallas kernel here")


# speedup vs baseline: 2.0780x; 2.0780x over previous
import jax
import jax.numpy as jnp
from jax.experimental import pallas as pl

_ENC = 128
_BS = 512


def _body(x_ref, e_ref, o_ref):
    d = x_ref.shape[-1]
    o_ref[:, :, :d] = x_ref[...]
    o_ref[:, :, d:] = e_ref[...][None, :, :]


def kernel(x, embed_table):
    b, s, d = x.shape
    e = embed_table.shape[-1]
    grid = (b, s // _BS)
    return pl.pallas_call(
        _body,
        grid=grid,
        in_specs=[
            pl.BlockSpec((1, _BS, d), lambda i, j: (i, j, 0)),
            pl.BlockSpec((_BS, e), lambda i, j: (j, 0)),
        ],
        out_specs=pl.BlockSpec((1, _BS, d + e), lambda i, j: (i, j, 0)),
        out_shape=jax.ShapeDtypeStruct((b, s, d + e), x.dtype),
    )(x, embed_table)
